# baseline (speedup 1.0000x reference)
import jax
import jax.numpy as jnp
from jax.experimental import pallas as pl
from jax.experimental.pallas import tpu as pltpu

N = 10000
D_IN = 128
D_OUT = 128
EPS = 1e-5
BM = 400


def _support_kernel(x_ref, w_ref, out_ref):
    s = jnp.dot(x_ref[...], w_ref[...], preferred_element_type=jnp.float32)
    out_ref[...] = s.astype(jnp.bfloat16)


def _spmm_kernel(adj_ref, support_ref, scale_ref, shift_ref, out_ref):
    acc = jnp.dot(adj_ref[...].astype(jnp.bfloat16), support_ref[...],
                  preferred_element_type=jnp.float32)
    out_ref[...] = jnp.maximum(acc * scale_ref[...] + shift_ref[...], 0.0)


def kernel(x, adj, W, b, bn_gamma, bn_beta):
    scale = (bn_gamma / jnp.sqrt(1.0 + EPS)).reshape(1, D_OUT)
    shift = (b * scale[0] + bn_beta).reshape(1, D_OUT)

    support = pl.pallas_call(
        _support_kernel,
        grid=(10,),
        in_specs=[
            pl.BlockSpec((N // 10, D_IN), lambda i: (i, 0)),
            pl.BlockSpec((D_IN, D_OUT), lambda i: (0, 0)),
        ],
        out_specs=pl.BlockSpec((N // 10, D_OUT), lambda i: (i, 0)),
        out_shape=jax.ShapeDtypeStruct((N, D_OUT), jnp.bfloat16),
        compiler_params=pltpu.CompilerParams(
            dimension_semantics=("parallel",),
        ),
    )(x, W)

    return pl.pallas_call(
        _spmm_kernel,
        grid=(N // BM,),
        in_specs=[
            pl.BlockSpec((BM, N), lambda i: (i, 0)),
            pl.BlockSpec((N, D_OUT), lambda i: (0, 0)),
            pl.BlockSpec((1, D_OUT), lambda i: (0, 0)),
            pl.BlockSpec((1, D_OUT), lambda i: (0, 0)),
        ],
        out_specs=pl.BlockSpec((BM, D_OUT), lambda i: (i, 0)),
        out_shape=jax.ShapeDtypeStruct((N, D_OUT), jnp.float32),
        compiler_params=pltpu.CompilerParams(
            dimension_semantics=("parallel",),
        ),
    )(adj, support, scale, shift)


# speedup vs baseline: 1.0411x; 1.0411x over previous
import jax
import jax.numpy as jnp
from jax.experimental import pallas as pl
from jax.experimental.pallas import tpu as pltpu

N = 10000
D_IN = 128
D_OUT = 128
EPS = 1e-5
BM = 200
NSPLIT = 2
NB = N // (BM * NSPLIT)


def _gcn_kernel(x_ref, adj_a_ref, adj_b_ref, w_ref, scale_ref, shift_ref,
                out_ref, support_ref):
    @pl.when(pl.program_id(0) == 0)
    def _():
        s = jnp.dot(x_ref[...], w_ref[...],
                    preferred_element_type=jnp.float32)
        support_ref[...] = s.astype(jnp.bfloat16)

    sup = support_ref[...]
    acc_a = jnp.dot(adj_a_ref[0].astype(jnp.bfloat16), sup,
                    preferred_element_type=jnp.float32)
    out_ref[0] = jnp.maximum(acc_a * scale_ref[...] + shift_ref[...], 0.0)
    acc_b = jnp.dot(adj_b_ref[0].astype(jnp.bfloat16), sup,
                    preferred_element_type=jnp.float32)
    out_ref[1] = jnp.maximum(acc_b * scale_ref[...] + shift_ref[...], 0.0)


def kernel(x, adj, W, b, bn_gamma, bn_beta):
    scale = (bn_gamma / jnp.sqrt(1.0 + EPS)).reshape(1, D_OUT)
    shift = (b * scale[0] + bn_beta).reshape(1, D_OUT)

    adj_r = adj.reshape(N // BM, BM, N)

    out = pl.pallas_call(
        _gcn_kernel,
        grid=(NB,),
        in_specs=[
            pl.BlockSpec((N, D_IN), lambda i: (0, 0)),
            pl.BlockSpec((1, BM, N), lambda i: (2 * i, 0, 0)),
            pl.BlockSpec((1, BM, N), lambda i: (2 * i + 1, 0, 0)),
            pl.BlockSpec((D_IN, D_OUT), lambda i: (0, 0)),
            pl.BlockSpec((1, D_OUT), lambda i: (0, 0)),
            pl.BlockSpec((1, D_OUT), lambda i: (0, 0)),
        ],
        out_specs=pl.BlockSpec((2, BM, D_OUT), lambda i: (i, 0, 0)),
        out_shape=jax.ShapeDtypeStruct((N // BM, BM, D_OUT), jnp.float32),
        scratch_shapes=[pltpu.VMEM((N, D_OUT), jnp.bfloat16)],
        compiler_params=pltpu.CompilerParams(
            dimension_semantics=("arbitrary",),
        ),
    )(x, adj_r, adj_r, W, scale, shift)
    return out.reshape(N, D_OUT)


# speedup vs baseline: 1.0549x; 1.0133x over previous
import jax
import jax.numpy as jnp
from jax.experimental import pallas as pl
from jax.experimental.pallas import tpu as pltpu

N = 10000
D_IN = 128
D_OUT = 128
EPS = 1e-5
BM = 400


def _gcn_kernel(x_ref, adj_ref, w_ref, scale_ref, shift_ref, out_ref,
                support_ref):
    @pl.when(pl.program_id(0) == 0)
    def _():
        s = jnp.dot(x_ref[...], w_ref[...],
                    preferred_element_type=jnp.float32)
        support_ref[...] = s.astype(jnp.bfloat16)

    acc = jnp.dot(adj_ref[...].astype(jnp.bfloat16), support_ref[...],
                  preferred_element_type=jnp.float32)
    out_ref[...] = jnp.maximum(acc * scale_ref[...] + shift_ref[...], 0.0)


def kernel(x, adj, W, b, bn_gamma, bn_beta):
    scale = (bn_gamma / jnp.sqrt(1.0 + EPS)).reshape(1, D_OUT)
    shift = (b * scale[0] + bn_beta).reshape(1, D_OUT)

    return pl.pallas_call(
        _gcn_kernel,
        grid=(N // BM,),
        in_specs=[
            pl.BlockSpec((N, D_IN), lambda i: (0, 0)),
            pl.BlockSpec((BM, N), lambda i: (i, 0)),
            pl.BlockSpec((D_IN, D_OUT), lambda i: (0, 0)),
            pl.BlockSpec((1, D_OUT), lambda i: (0, 0)),
            pl.BlockSpec((1, D_OUT), lambda i: (0, 0)),
        ],
        out_specs=pl.BlockSpec((BM, D_OUT), lambda i: (i, 0)),
        out_shape=jax.ShapeDtypeStruct((N, D_OUT), jnp.float32),
        scratch_shapes=[pltpu.VMEM((N, D_OUT), jnp.bfloat16)],
        compiler_params=pltpu.CompilerParams(
            dimension_semantics=("arbitrary",),
        ),
    )(x, adj, W, scale, shift)


# speedup vs baseline: 1.0640x; 1.0086x over previous
import jax
import jax.numpy as jnp
from jax.experimental import pallas as pl
from jax.experimental.pallas import tpu as pltpu
from jax.experimental.shard_map import shard_map
from jax.sharding import Mesh, PartitionSpec as P

N = 10000
D_IN = 128
D_OUT = 128
EPS = 1e-5


def _gcn_kernel(x_ref, adj_ref, w_ref, scale_ref, shift_ref, out_ref,
                support_ref):
    @pl.when(pl.program_id(0) == 0)
    def _():
        s = jnp.dot(x_ref[...], w_ref[...],
                    preferred_element_type=jnp.float32)
        support_ref[...] = s.astype(jnp.bfloat16)

    acc = jnp.dot(adj_ref[...].astype(jnp.bfloat16), support_ref[...],
                  preferred_element_type=jnp.float32)
    out_ref[...] = jnp.maximum(acc * scale_ref[...] + shift_ref[...], 0.0)


def _local_gcn(x, adj_shard, W, scale, shift):
    rows = adj_shard.shape[0]
    bm = 400 if rows % 400 == 0 else 200
    return pl.pallas_call(
        _gcn_kernel,
        grid=(rows // bm,),
        in_specs=[
            pl.BlockSpec((N, D_IN), lambda i: (0, 0)),
            pl.BlockSpec((bm, N), lambda i: (i, 0)),
            pl.BlockSpec((D_IN, D_OUT), lambda i: (0, 0)),
            pl.BlockSpec((1, D_OUT), lambda i: (0, 0)),
            pl.BlockSpec((1, D_OUT), lambda i: (0, 0)),
        ],
        out_specs=pl.BlockSpec((bm, D_OUT), lambda i: (i, 0)),
        out_shape=jax.ShapeDtypeStruct((rows, D_OUT), jnp.float32),
        scratch_shapes=[pltpu.VMEM((N, D_OUT), jnp.bfloat16)],
        compiler_params=pltpu.CompilerParams(
            dimension_semantics=("arbitrary",),
        ),
    )(x, adj_shard, W, scale, shift)


def kernel(x, adj, W, b, bn_gamma, bn_beta):
    scale = (bn_gamma / jnp.sqrt(1.0 + EPS)).reshape(1, D_OUT)
    shift = (b * scale[0] + bn_beta).reshape(1, D_OUT)

    devices = jax.devices()
    n_shards = 2 if len(devices) >= 2 and N % (2 * 400) == 0 else 1
    if n_shards == 1:
        return _local_gcn(x, adj, W, scale, shift)

    mesh = Mesh(devices[:n_shards], ("i",))
    sharded = shard_map(
        _local_gcn,
        mesh=mesh,
        in_specs=(P(), P("i", None), P(), P(), P()),
        out_specs=P("i", None),
    )
    return sharded(x, adj, W, scale, shift)
